# baseline (speedup 1.0000x reference)
import jax
import jax.numpy as jnp
from jax.experimental import pallas as pl
from jax.experimental.pallas import tpu as pltpu

_N_BINS = 10
_W = 1024


def _ce_kernel(xt_ref, labels_ref, lo_ref, hi_ref, out_ref,
               cnt_ref, conf_ref, acc_ref):
    i = pl.program_id(0)
    nsteps = pl.num_programs(0)

    @pl.when(i == 0)
    def _init():
        cnt_ref[...] = jnp.zeros_like(cnt_ref)
        conf_ref[...] = jnp.zeros_like(conf_ref)
        acc_ref[...] = jnp.zeros_like(acc_ref)

    x = xt_ref[...]
    c, w = x.shape
    m = jnp.max(x, axis=0, keepdims=True)
    iota = jax.lax.broadcasted_iota(jnp.int32, (c, w), 0).astype(jnp.float32)
    idx = jnp.min(jnp.where(x == m, iota, jnp.float32(c)),
                  axis=0, keepdims=True)
    lab = labels_ref[0].astype(jnp.float32)
    correct = (idx == lab).astype(jnp.float32)

    lo = lo_ref[...]
    hi = hi_ref[...]
    mask = ((m > lo) & (m <= hi)).astype(jnp.float32)
    cnt_ref[...] += mask
    conf_ref[...] += mask * m
    acc_ref[...] += mask * correct

    @pl.when(i == nsteps - 1)
    def _finish():
        cnt = jnp.sum(cnt_ref[...], axis=1, keepdims=True)
        conf = jnp.sum(conf_ref[...], axis=1, keepdims=True)
        acc = jnp.sum(acc_ref[...], axis=1, keepdims=True)
        total = jnp.sum(cnt)
        valid = (cnt > 0).astype(jnp.float32)
        denom = jnp.maximum(cnt, 1.0)
        terms = (cnt / total) * (conf / denom - acc / denom) ** 2 * valid
        out_ref[...] = jnp.sqrt(
            jnp.broadcast_to(jnp.sum(terms), (1, 1)))


def kernel(probas, labels):
    n, c = probas.shape
    xt = probas.T
    nb = n // _W
    labels3 = labels.reshape(nb, 1, _W)
    bins = jnp.linspace(0.0, 1.0, _N_BINS + 1)
    pad = jnp.full((16 - _N_BINS,), 2.0, jnp.float32)
    lo = jnp.concatenate([bins[:-1], pad]).reshape(16, 1)
    hi = jnp.concatenate([bins[1:], pad]).reshape(16, 1)

    out = pl.pallas_call(
        _ce_kernel,
        grid=(nb,),
        in_specs=[
            pl.BlockSpec((c, _W), lambda i: (0, i)),
            pl.BlockSpec((1, 1, _W), lambda i: (i, 0, 0)),
            pl.BlockSpec((16, 1), lambda i: (0, 0)),
            pl.BlockSpec((16, 1), lambda i: (0, 0)),
        ],
        out_specs=pl.BlockSpec((1, 1), lambda i: (0, 0)),
        out_shape=jax.ShapeDtypeStruct((1, 1), jnp.float32),
        scratch_shapes=[
            pltpu.VMEM((16, _W), jnp.float32),
            pltpu.VMEM((16, _W), jnp.float32),
            pltpu.VMEM((16, _W), jnp.float32),
        ],
        compiler_params=pltpu.CompilerParams(
            dimension_semantics=("arbitrary",),
        ),
    )(xt, labels3, lo, hi)
    return out[0, 0]


# speedup vs baseline: 1.1198x; 1.1198x over previous
import jax
import jax.numpy as jnp
from jax.experimental import pallas as pl
from jax.experimental.pallas import tpu as pltpu

_N_BINS = 10
_W = 2048


def _ce_kernel(xt_ref, labels_ref, lo_ref, hi_ref, out_ref,
               cnt_ref, conf_ref, acc_ref):
    i = pl.program_id(0)
    nsteps = pl.num_programs(0)

    @pl.when(i == 0)
    def _init():
        cnt_ref[...] = jnp.zeros_like(cnt_ref)
        conf_ref[...] = jnp.zeros_like(conf_ref)
        acc_ref[...] = jnp.zeros_like(acc_ref)

    x = xt_ref[...]
    c, w = x.shape
    m = jnp.max(x, axis=0, keepdims=True)
    iota = jax.lax.broadcasted_iota(jnp.int32, (c, w), 0).astype(jnp.float32)
    idx = jnp.min(jnp.where(x == m, iota, jnp.float32(c)),
                  axis=0, keepdims=True)
    lab = labels_ref[0].astype(jnp.float32)
    correct = (idx == lab).astype(jnp.float32)

    lo = lo_ref[...]
    hi = hi_ref[...]
    mask = ((m > lo) & (m <= hi)).astype(jnp.float32)
    cnt_ref[...] += mask
    conf_ref[...] += mask * m
    acc_ref[...] += mask * correct

    @pl.when(i == nsteps - 1)
    def _finish():
        cnt = jnp.sum(cnt_ref[...], axis=1, keepdims=True)
        conf = jnp.sum(conf_ref[...], axis=1, keepdims=True)
        acc = jnp.sum(acc_ref[...], axis=1, keepdims=True)
        total = jnp.sum(cnt)
        valid = (cnt > 0).astype(jnp.float32)
        denom = jnp.maximum(cnt, 1.0)
        terms = (cnt / total) * (conf / denom - acc / denom) ** 2 * valid
        out_ref[...] = jnp.sqrt(
            jnp.broadcast_to(jnp.sum(terms), (1, 1)))


def kernel(probas, labels):
    n, c = probas.shape
    xt = probas.T
    nb = n // _W
    labels3 = labels.reshape(nb, 1, _W)
    bins = jnp.linspace(0.0, 1.0, _N_BINS + 1)
    pad = jnp.full((16 - _N_BINS,), 2.0, jnp.float32)
    lo = jnp.concatenate([bins[:-1], pad]).reshape(16, 1)
    hi = jnp.concatenate([bins[1:], pad]).reshape(16, 1)

    out = pl.pallas_call(
        _ce_kernel,
        grid=(nb,),
        in_specs=[
            pl.BlockSpec((c, _W), lambda i: (0, i)),
            pl.BlockSpec((1, 1, _W), lambda i: (i, 0, 0)),
            pl.BlockSpec((16, 1), lambda i: (0, 0)),
            pl.BlockSpec((16, 1), lambda i: (0, 0)),
        ],
        out_specs=pl.BlockSpec((1, 1), lambda i: (0, 0)),
        out_shape=jax.ShapeDtypeStruct((1, 1), jnp.float32),
        scratch_shapes=[
            pltpu.VMEM((16, _W), jnp.float32),
            pltpu.VMEM((16, _W), jnp.float32),
            pltpu.VMEM((16, _W), jnp.float32),
        ],
        compiler_params=pltpu.CompilerParams(
            dimension_semantics=("arbitrary",),
        ),
    )(xt, labels3, lo, hi)
    return out[0, 0]
